# Initial kernel scaffold; baseline (speedup 1.0000x reference)
import jax
import jax.numpy as jnp
from jax.experimental import pallas as pl


def kernel(states, action, We1, be1, We2, be2, ge, bne, We3, be3, Wn1, bn1, Wn2, bn2, gn, bnn, Wn3, bn3):
    raise NotImplementedError("write your pallas kernel here")


# speedup vs baseline: 7.8197x; 7.8197x over previous
import jax
import jax.numpy as jnp
from jax.experimental import pallas as pl
from jax.experimental.pallas import tpu as pltpu

_B = 512
_O = 5
_OBS = 128
_HID = 256
_EPN = _O - 1
_F = 4 * _HID

_BB = 64
_R = _BB * _O
_RE = _R * _EPN


def _c4_flat(W):
    g = jnp.arange(4)[:, None]
    h = jnp.arange(4)[None, :]
    Wfull = W[(g - h) % 4]
    Wt = jnp.transpose(Wfull, (1, 2, 0, 3))
    return Wt.reshape(4 * W.shape[1], 4 * W.shape[2])


def _ln_relu(y, gamma, beta):
    outs = []
    for gi in range(4):
        c = y[:, gi * _HID:(gi + 1) * _HID]
        mu = jnp.mean(c, axis=1, keepdims=True)
        d = c - mu
        var = jnp.mean(d * d, axis=1, keepdims=True)
        outs.append(d * jax.lax.rsqrt(var + 1e-5) * gamma + beta)
    return jnp.maximum(jnp.concatenate(outs, axis=1), 0.0)


def _dot(a, b):
    return jnp.dot(a, b, preferred_element_type=jnp.float32)


def _body(x_ref, act_ref, w1s_ref, w1t_ref, b1_ref, w2_ref, b2_ref,
          ge_ref, bne_ref, w3_ref, b3_ref, wn1o_ref, wav_ref, wn1a_ref,
          bn1_ref, wn2_ref, bn2_ref, gn_ref, bnn_ref, wn3_ref, bn3_ref,
          out_ref):
    x = x_ref[...]

    A = _dot(x, w1s_ref[...])
    T = _dot(x, w1t_ref[...])
    A3 = A.reshape(_BB, _O, _F)
    T3 = T.reshape(_BB, _O, _F)

    nbs = []
    for i in range(_O):
        if i == 0:
            nb = T3[:, 1:]
        elif i == _O - 1:
            nb = T3[:, :_O - 1]
        else:
            nb = jnp.concatenate([T3[:, :i], T3[:, i + 1:]], axis=1)
        nbs.append(nb)
    Tg = jnp.stack(nbs, axis=1)

    e = A3[:, :, None, :] + Tg + b1_ref[...].reshape(1, 1, 1, _F)
    e = jnp.maximum(e, 0.0).reshape(_RE, _F)

    e = _dot(e, w2_ref[...]) + b2_ref[...]
    e = _ln_relu(e, ge_ref[...], bne_ref[...])
    e = _dot(e, w3_ref[...]) + b3_ref[...]

    agg = jnp.sum(e.reshape(_BB, _O, _EPN, _F), axis=2)
    aggR = agg.reshape(_R, _F)

    act = act_ref[0, 0, :].reshape(_BB, 1)
    ii = jax.lax.broadcasted_iota(jnp.int32, (_BB, _O), 1)
    avt = jnp.zeros((_BB, _O, _F), jnp.float32)
    wav = wav_ref[...]
    for h in range(4):
        m = (act == ii * 4 + h).astype(jnp.float32)[:, :, None]
        avt = avt + m * wav[h:h + 1, :].reshape(1, 1, _F)

    n = (_dot(x, wn1o_ref[...]) + avt.reshape(_R, _F)
         + _dot(aggR, wn1a_ref[...]) + bn1_ref[...])
    n = jnp.maximum(n, 0.0)
    n = _dot(n, wn2_ref[...]) + bn2_ref[...]
    n = _ln_relu(n, gn_ref[...], bnn_ref[...])
    out_ref[...] = _dot(n, wn3_ref[...]) + bn3_ref[...]


def kernel(states, action, We1, be1, We2, be2, ge, bne, We3, be3,
           Wn1, bn1, Wn2, bn2, gn, bnn, Wn3, bn3):
    x = states.reshape(_B * _O, 4 * _OBS)
    nblk = _B // _BB
    act = action.astype(jnp.int32).reshape(nblk, 1, _BB)

    W1s = _c4_flat(We1[:, :_OBS, :])
    W1t = _c4_flat(We1[:, _OBS:, :])
    W2 = _c4_flat(We2)
    W3 = _c4_flat(We3)
    Wn1o = _c4_flat(Wn1[:, :_OBS, :])
    Wav = _c4_flat(Wn1[:, _OBS:_OBS + 1, :])
    Wn1a = _c4_flat(Wn1[:, _OBS + 1:, :])
    Wn2f = _c4_flat(Wn2)
    Wn3f = _c4_flat(Wn3)

    b1 = jnp.tile(be1, 4).reshape(1, _F)
    b2 = jnp.tile(be2, 4).reshape(1, _F)
    b3 = jnp.tile(be3, 4).reshape(1, _F)
    bn1r = jnp.tile(bn1, 4).reshape(1, _F)
    bn2r = jnp.tile(bn2, 4).reshape(1, _F)
    bn3r = jnp.tile(bn3, 4).reshape(1, 4 * _OBS)
    ge2 = ge.reshape(1, _HID)
    bne2 = bne.reshape(1, _HID)
    gn2 = gn.reshape(1, _HID)
    bnn2 = bnn.reshape(1, _HID)

    def const_spec(a):
        nd = a.ndim
        return pl.BlockSpec(a.shape, lambda i, _nd=nd: (0,) * _nd)

    weights = (W1s, W1t, b1, W2, b2, ge2, bne2, W3, b3,
               Wn1o, Wav, Wn1a, bn1r, Wn2f, bn2r, gn2, bnn2, Wn3f, bn3r)

    out = pl.pallas_call(
        _body,
        grid=(nblk,),
        in_specs=[
            pl.BlockSpec((_R, 4 * _OBS), lambda i: (i, 0)),
            pl.BlockSpec((1, 1, _BB), lambda i: (i, 0, 0)),
        ] + [const_spec(w) for w in weights],
        out_specs=pl.BlockSpec((_R, 4 * _OBS), lambda i: (i, 0)),
        out_shape=jax.ShapeDtypeStruct((_B * _O, 4 * _OBS), jnp.float32),
        compiler_params=pltpu.CompilerParams(
            dimension_semantics=("arbitrary",)),
    )(x, act, *weights)

    return out.reshape(_B, _O, 4, _OBS)


# speedup vs baseline: 8.6510x; 1.1063x over previous
import jax
import jax.numpy as jnp
from jax.experimental import pallas as pl
from jax.experimental.pallas import tpu as pltpu

_B = 512
_O = 5
_OBS = 128
_HID = 256
_EPN = _O - 1
_F = 4 * _HID

_BB = 64
_R = _BB * _O
_RE = _R * _EPN


def _c4_flat(W):
    g = jnp.arange(4)[:, None]
    h = jnp.arange(4)[None, :]
    Wfull = W[(g - h) % 4]
    Wt = jnp.transpose(Wfull, (1, 2, 0, 3))
    return Wt.reshape(4 * W.shape[1], 4 * W.shape[2])


def _ln_relu(y, gamma, beta):
    outs = []
    for gi in range(4):
        c = y[:, gi * _HID:(gi + 1) * _HID]
        mu = jnp.mean(c, axis=1, keepdims=True)
        d = c - mu
        var = jnp.mean(d * d, axis=1, keepdims=True)
        outs.append(d * jax.lax.rsqrt(var + 1e-5) * gamma + beta)
    return jnp.maximum(jnp.concatenate(outs, axis=1), 0.0)


def _dot(a, b):
    return jnp.dot(a.astype(jnp.bfloat16), b,
                   preferred_element_type=jnp.float32)


def _body(x_ref, act_ref, w1s_ref, w1t_ref, b1_ref, w2_ref, b2_ref,
          ge_ref, bne_ref, w3_ref, b3_ref, wn1o_ref, wav_ref, wn1a_ref,
          bn1_ref, wn2_ref, bn2_ref, gn_ref, bnn_ref, wn3_ref, bn3_ref,
          out_ref):
    x = x_ref[...]

    A = _dot(x, w1s_ref[...])
    T = _dot(x, w1t_ref[...])
    A3 = A.reshape(_BB, _O, _F)
    T3 = T.reshape(_BB, _O, _F)

    nbs = []
    for i in range(_O):
        if i == 0:
            nb = T3[:, 1:]
        elif i == _O - 1:
            nb = T3[:, :_O - 1]
        else:
            nb = jnp.concatenate([T3[:, :i], T3[:, i + 1:]], axis=1)
        nbs.append(nb)
    Tg = jnp.stack(nbs, axis=1)

    e = A3[:, :, None, :] + Tg + b1_ref[...].reshape(1, 1, 1, _F)
    e = jnp.maximum(e, 0.0).reshape(_RE, _F)

    e = _dot(e, w2_ref[...]) + b2_ref[...]
    e = _ln_relu(e, ge_ref[...], bne_ref[...])
    e = _dot(e, w3_ref[...]) + b3_ref[...]

    agg = jnp.sum(e.reshape(_BB, _O, _EPN, _F), axis=2)
    aggR = agg.reshape(_R, _F)

    act = act_ref[0, 0, :].reshape(_BB, 1)
    ii = jax.lax.broadcasted_iota(jnp.int32, (_BB, _O), 1)
    avt = jnp.zeros((_BB, _O, _F), jnp.float32)
    wav = wav_ref[...]
    for h in range(4):
        m = (act == ii * 4 + h).astype(jnp.float32)[:, :, None]
        avt = avt + m * wav[h:h + 1, :].reshape(1, 1, _F)

    n = (_dot(x, wn1o_ref[...]) + avt.reshape(_R, _F)
         + _dot(aggR, wn1a_ref[...]) + bn1_ref[...])
    n = jnp.maximum(n, 0.0)
    n = _dot(n, wn2_ref[...]) + bn2_ref[...]
    n = _ln_relu(n, gn_ref[...], bnn_ref[...])
    out_ref[...] = _dot(n, wn3_ref[...]) + bn3_ref[...]


def kernel(states, action, We1, be1, We2, be2, ge, bne, We3, be3,
           Wn1, bn1, Wn2, bn2, gn, bnn, Wn3, bn3):
    x = states.reshape(_B * _O, 4 * _OBS)
    nblk = _B // _BB
    act = action.astype(jnp.int32).reshape(nblk, 1, _BB)

    bf16 = jnp.bfloat16
    W1s = _c4_flat(We1[:, :_OBS, :]).astype(bf16)
    W1t = _c4_flat(We1[:, _OBS:, :]).astype(bf16)
    W2 = _c4_flat(We2).astype(bf16)
    W3 = _c4_flat(We3).astype(bf16)
    Wn1o = _c4_flat(Wn1[:, :_OBS, :]).astype(bf16)
    Wav = _c4_flat(Wn1[:, _OBS:_OBS + 1, :])
    Wn1a = _c4_flat(Wn1[:, _OBS + 1:, :]).astype(bf16)
    Wn2f = _c4_flat(Wn2).astype(bf16)
    Wn3f = _c4_flat(Wn3).astype(bf16)

    b1 = jnp.tile(be1, 4).reshape(1, _F)
    b2 = jnp.tile(be2, 4).reshape(1, _F)
    b3 = jnp.tile(be3, 4).reshape(1, _F)
    bn1r = jnp.tile(bn1, 4).reshape(1, _F)
    bn2r = jnp.tile(bn2, 4).reshape(1, _F)
    bn3r = jnp.tile(bn3, 4).reshape(1, 4 * _OBS)
    ge2 = ge.reshape(1, _HID)
    bne2 = bne.reshape(1, _HID)
    gn2 = gn.reshape(1, _HID)
    bnn2 = bnn.reshape(1, _HID)

    def const_spec(a):
        nd = a.ndim
        return pl.BlockSpec(a.shape, lambda i, _nd=nd: (0,) * _nd)

    weights = (W1s, W1t, b1, W2, b2, ge2, bne2, W3, b3,
               Wn1o, Wav, Wn1a, bn1r, Wn2f, bn2r, gn2, bnn2, Wn3f, bn3r)

    out = pl.pallas_call(
        _body,
        grid=(nblk,),
        in_specs=[
            pl.BlockSpec((_R, 4 * _OBS), lambda i: (i, 0)),
            pl.BlockSpec((1, 1, _BB), lambda i: (i, 0, 0)),
        ] + [const_spec(w) for w in weights],
        out_specs=pl.BlockSpec((_R, 4 * _OBS), lambda i: (i, 0)),
        out_shape=jax.ShapeDtypeStruct((_B * _O, 4 * _OBS), jnp.float32),
        compiler_params=pltpu.CompilerParams(
            dimension_semantics=("arbitrary",)),
    )(x, act, *weights)

    return out.reshape(_B, _O, 4, _OBS)


# speedup vs baseline: 10.3308x; 1.1942x over previous
import numpy as np
import jax
import jax.numpy as jnp
from jax.experimental import pallas as pl
from jax.experimental.pallas import tpu as pltpu

_B = 512
_O = 5
_OBS = 128
_HID = 256
_EPN = _O - 1
_F = 4 * _HID
_EPS = _O * _EPN

_BB = 64
_R = _BB * _O
_RE = _BB * _EPS


def _sel_matrices():
    s0 = np.zeros((_EPS, _O), np.float32)
    c0 = np.zeros((_EPS, _O), np.float32)
    e = 0
    for i in range(_O):
        for j in range(_O):
            if i == j:
                continue
            s0[e, i] = 1.0
            c0[e, j] = 1.0
            e += 1
    eye = np.eye(_BB, dtype=np.float32)
    S = np.concatenate([np.kron(eye, s0), np.kron(eye, c0)], axis=1)
    Ragg = np.kron(eye, s0.T)
    return S, Ragg


_S_NP, _RAGG_NP = _sel_matrices()


def _c4_flat(W):
    g = jnp.arange(4)[:, None]
    h = jnp.arange(4)[None, :]
    Wfull = W[(g - h) % 4]
    Wt = jnp.transpose(Wfull, (1, 2, 0, 3))
    return Wt.reshape(4 * W.shape[1], 4 * W.shape[2])


def _ln_relu(y, gamma, beta):
    outs = []
    for gi in range(4):
        c = y[:, gi * _HID:(gi + 1) * _HID]
        mu = jnp.mean(c, axis=1, keepdims=True)
        d = c - mu
        var = jnp.mean(d * d, axis=1, keepdims=True)
        outs.append(d * jax.lax.rsqrt(var + 1e-5) * gamma + beta)
    return jnp.maximum(jnp.concatenate(outs, axis=1), 0.0)


def _dot(a, b):
    return jnp.dot(a.astype(jnp.bfloat16), b,
                   preferred_element_type=jnp.float32)


def _body(x_ref, act_ref, sel_ref, ragg_ref, w1s_ref, w1t_ref, b1_ref,
          w2_ref, b2_ref, ge_ref, bne_ref, w3_ref, b3_ref, wn1o_ref,
          wav_ref, wn1a_ref, bn1_ref, wn2_ref, bn2_ref, gn_ref, bnn_ref,
          wn3_ref, bn3_ref, out_ref):
    x = x_ref[...]

    A = _dot(x, w1s_ref[...])
    T = _dot(x, w1t_ref[...])
    AT = jnp.concatenate([A, T], axis=0)

    e = _dot(sel_ref[...], AT) + b1_ref[...]
    e = jnp.maximum(e, 0.0)

    e = _dot(e, w2_ref[...]) + b2_ref[...]
    e = _ln_relu(e, ge_ref[...], bne_ref[...])
    e = _dot(e, w3_ref[...]) + b3_ref[...]

    agg = _dot(ragg_ref[...], e)

    act = act_ref[0, 0, :].reshape(_R, 1)
    rr = jax.lax.broadcasted_iota(jnp.int32, (_R, 4), 0)
    hh = jax.lax.broadcasted_iota(jnp.int32, (_R, 4), 1)
    M = (act == 4 * (rr % _O) + hh).astype(jnp.float32)

    n = (_dot(x, wn1o_ref[...]) + _dot(M, wav_ref[...])
         + _dot(agg, wn1a_ref[...]) + bn1_ref[...])
    n = jnp.maximum(n, 0.0)
    n = _dot(n, wn2_ref[...]) + bn2_ref[...]
    n = _ln_relu(n, gn_ref[...], bnn_ref[...])
    out_ref[...] = _dot(n, wn3_ref[...]) + bn3_ref[...]


def kernel(states, action, We1, be1, We2, be2, ge, bne, We3, be3,
           Wn1, bn1, Wn2, bn2, gn, bnn, Wn3, bn3):
    x = states.reshape(_B * _O, 4 * _OBS)
    nblk = _B // _BB
    act = jnp.repeat(action.astype(jnp.int32), _O).reshape(nblk, 1, _R)

    bf16 = jnp.bfloat16
    S = jnp.asarray(_S_NP, dtype=bf16)
    Ragg = jnp.asarray(_RAGG_NP, dtype=bf16)
    W1s = _c4_flat(We1[:, :_OBS, :]).astype(bf16)
    W1t = _c4_flat(We1[:, _OBS:, :]).astype(bf16)
    W2 = _c4_flat(We2).astype(bf16)
    W3 = _c4_flat(We3).astype(bf16)
    Wn1o = _c4_flat(Wn1[:, :_OBS, :]).astype(bf16)
    Wav = _c4_flat(Wn1[:, _OBS:_OBS + 1, :]).astype(bf16)
    Wn1a = _c4_flat(Wn1[:, _OBS + 1:, :]).astype(bf16)
    Wn2f = _c4_flat(Wn2).astype(bf16)
    Wn3f = _c4_flat(Wn3).astype(bf16)

    b1 = jnp.tile(be1, 4).reshape(1, _F)
    b2 = jnp.tile(be2, 4).reshape(1, _F)
    b3 = jnp.tile(be3, 4).reshape(1, _F)
    bn1r = jnp.tile(bn1, 4).reshape(1, _F)
    bn2r = jnp.tile(bn2, 4).reshape(1, _F)
    bn3r = jnp.tile(bn3, 4).reshape(1, 4 * _OBS)
    ge2 = ge.reshape(1, _HID)
    bne2 = bne.reshape(1, _HID)
    gn2 = gn.reshape(1, _HID)
    bnn2 = bnn.reshape(1, _HID)

    def const_spec(a):
        nd = a.ndim
        return pl.BlockSpec(a.shape, lambda i, _nd=nd: (0,) * _nd)

    weights = (S, Ragg, W1s, W1t, b1, W2, b2, ge2, bne2, W3, b3,
               Wn1o, Wav, Wn1a, bn1r, Wn2f, bn2r, gn2, bnn2, Wn3f, bn3r)

    out = pl.pallas_call(
        _body,
        grid=(nblk,),
        in_specs=[
            pl.BlockSpec((_R, 4 * _OBS), lambda i: (i, 0)),
            pl.BlockSpec((1, 1, _R), lambda i: (i, 0, 0)),
        ] + [const_spec(w) for w in weights],
        out_specs=pl.BlockSpec((_R, 4 * _OBS), lambda i: (i, 0)),
        out_shape=jax.ShapeDtypeStruct((_B * _O, 4 * _OBS), jnp.float32),
        compiler_params=pltpu.CompilerParams(
            dimension_semantics=("arbitrary",)),
    )(x, act, *weights)

    return out.reshape(_B, _O, 4, _OBS)
